# baseline (speedup 1.0000x reference)
import jax
import jax.numpy as jnp
from jax import lax
from jax.experimental import pallas as pl
from jax.experimental.pallas import tpu as pltpu
from jax.experimental.pallas import tpu_sc as plsc

_NC = 2
_NS = 16
_NW = _NC * _NS

_D = 32
_N_ROWS = 1_000_000
_B = 4096 * 200
_B_PER_W = _B // _NW
_CHUNK = 256
_NCHUNK = _B_PER_W // _CHUNK
_NBUF = 4
_IDX_HEAD = _B_PER_W // 2


def _body(table_hbm, idx_hbm, out_hbm, idx_v, rows_v, gsems, osems, isem):
    wid = lax.axis_index("s") * _NC + lax.axis_index("c")
    base = wid * _B_PER_W
    pltpu.sync_copy(idx_hbm.at[pl.ds(base, _IDX_HEAD)],
                    idx_v.at[pl.ds(0, _IDX_HEAD)])
    idx_tail = pltpu.async_copy(
        idx_hbm.at[pl.ds(base + _IDX_HEAD, _B_PER_W - _IDX_HEAD)],
        idx_v.at[pl.ds(_IDX_HEAD, _B_PER_W - _IDX_HEAD)], isem)

    def start_gather(c):
        b = c % _NBUF
        return pltpu.async_copy(
            table_hbm.at[idx_v.at[pl.ds(c * _CHUNK, _CHUNK)]],
            rows_v.at[b], gsems[b])

    def start_out(c):
        b = c % _NBUF
        return pltpu.async_copy(
            rows_v.at[b], out_hbm.at[pl.ds(base + c * _CHUNK, _CHUNK)],
            osems[b])

    tail_chunk = _IDX_HEAD // _CHUNK
    waited = [False]

    def gather_guarded(c):
        if c >= tail_chunk and not waited[0]:
            idx_tail.wait()
            waited[0] = True
        return start_gather(c)

    gds = [gather_guarded(c) for c in range(_NBUF)]
    ods = [None] * _NCHUNK
    for c in range(_NCHUNK):
        gds[c % _NBUF].wait()
        ods[c] = start_out(c)
        p = c - 1
        if p >= 0 and p + _NBUF < _NCHUNK:
            ods[p].wait()
            gds[p % _NBUF] = gather_guarded(p + _NBUF)
    for c in range(_NCHUNK - _NBUF, _NCHUNK):
        ods[c].wait()


@jax.jit
def kernel(indices, table):
    flat_idx = indices.reshape(_B)
    mesh = plsc.VectorSubcoreMesh(core_axis_name="c", subcore_axis_name="s")
    out = pl.kernel(
        _body,
        out_type=jax.ShapeDtypeStruct((_B, _D), jnp.float32),
        mesh=mesh,
        scratch_types=[
            pltpu.VMEM((_B_PER_W,), jnp.int32),
            pltpu.VMEM((_NBUF, _CHUNK, _D), jnp.float32),
            [pltpu.SemaphoreType.DMA] * _NBUF,
            [pltpu.SemaphoreType.DMA] * _NBUF,
            pltpu.SemaphoreType.DMA,
        ],
        compiler_params=pltpu.CompilerParams(use_tc_tiling_on_sc=False),
    )(table, flat_idx)
    return out.reshape(indices.shape[0], indices.shape[1], _D)


# speedup vs baseline: 1.0034x; 1.0034x over previous
import jax
import jax.numpy as jnp
from jax import lax
from jax.experimental import pallas as pl
from jax.experimental.pallas import tpu as pltpu
from jax.experimental.pallas import tpu_sc as plsc

_NC = 2
_NS = 16
_NW = _NC * _NS

_D = 32
_N_ROWS = 1_000_000
_B = 4096 * 200
_B_PER_W = _B // _NW
_CHUNK = 512
_NCHUNK = _B_PER_W // _CHUNK
_NBUF = 6
_IDX_HEAD = _B_PER_W // 2


def _body(table_hbm, idx_hbm, out_hbm, idx_v, rows_v, gsems, osems, isem):
    wid = lax.axis_index("s") * _NC + lax.axis_index("c")
    base = wid * _B_PER_W
    pltpu.sync_copy(idx_hbm.at[pl.ds(base, _IDX_HEAD)],
                    idx_v.at[pl.ds(0, _IDX_HEAD)])
    idx_tail = pltpu.async_copy(
        idx_hbm.at[pl.ds(base + _IDX_HEAD, _B_PER_W - _IDX_HEAD)],
        idx_v.at[pl.ds(_IDX_HEAD, _B_PER_W - _IDX_HEAD)], isem)

    def start_gather(c):
        b = c % _NBUF
        return pltpu.async_copy(
            table_hbm.at[idx_v.at[pl.ds(c * _CHUNK, _CHUNK)]],
            rows_v.at[b], gsems[b])

    def start_out(c):
        b = c % _NBUF
        return pltpu.async_copy(
            rows_v.at[b], out_hbm.at[pl.ds(base + c * _CHUNK, _CHUNK)],
            osems[b])

    tail_chunk = _IDX_HEAD // _CHUNK
    waited = [False]

    def gather_guarded(c):
        if c >= tail_chunk and not waited[0]:
            idx_tail.wait()
            waited[0] = True
        return start_gather(c)

    gds = [gather_guarded(c) for c in range(_NBUF)]
    ods = [None] * _NCHUNK
    for c in range(_NCHUNK):
        gds[c % _NBUF].wait()
        ods[c] = start_out(c)
        p = c - 1
        if p >= 0 and p + _NBUF < _NCHUNK:
            ods[p].wait()
            gds[p % _NBUF] = gather_guarded(p + _NBUF)
    for c in range(_NCHUNK - _NBUF, _NCHUNK):
        ods[c].wait()


@jax.jit
def kernel(indices, table):
    flat_idx = indices.reshape(_B)
    mesh = plsc.VectorSubcoreMesh(core_axis_name="c", subcore_axis_name="s")
    out = pl.kernel(
        _body,
        out_type=jax.ShapeDtypeStruct((_B, _D), jnp.float32),
        mesh=mesh,
        scratch_types=[
            pltpu.VMEM((_B_PER_W,), jnp.int32),
            pltpu.VMEM((_NBUF, _CHUNK, _D), jnp.float32),
            [pltpu.SemaphoreType.DMA] * _NBUF,
            [pltpu.SemaphoreType.DMA] * _NBUF,
            pltpu.SemaphoreType.DMA,
        ],
        compiler_params=pltpu.CompilerParams(use_tc_tiling_on_sc=False),
    )(table, flat_idx)
    return out.reshape(indices.shape[0], indices.shape[1], _D)
